# baseline (speedup 1.0000x reference)
import functools

import jax
import jax.numpy as jnp
from jax import lax
from jax.experimental import pallas as pl
from jax.experimental.pallas import tpu as pltpu
from jax.experimental.pallas import tpu_sc as plsc

ROW = 272
BATCH = 16384
NUM_CORES = 2
NUM_SUBCORES = 16
NW = NUM_CORES * NUM_SUBCORES
B_PER_W = BATCH // NW
CHUNK = 128
N_CHUNKS = B_PER_W // CHUNK
NBUF = 2

_mesh = plsc.VectorSubcoreMesh(core_axis_name="c", subcore_axis_name="s")


@functools.partial(
    pl.kernel,
    mesh=_mesh,
    out_type=jax.ShapeDtypeStruct((BATCH, ROW), jnp.float32),
    scratch_types=[
        pltpu.VMEM((N_CHUNKS, CHUNK), jnp.int32),
        pltpu.VMEM((NBUF, CHUNK, ROW), jnp.float32),
        pltpu.SemaphoreType.DMA,
        pltpu.SemaphoreType.DMA,
    ],
    compiler_params=pltpu.CompilerParams(use_tc_tiling_on_sc=False),
)
def _gather_kernel(idx_hbm, table_hbm, out_hbm, idx_v, rows_v, gsem, wsem):
    wid = lax.axis_index("s") * NUM_CORES + lax.axis_index("c")
    base = wid * B_PER_W
    pltpu.sync_copy(idx_hbm.at[wid], idx_v)

    gathers = [None] * N_CHUNKS
    writes = [None] * N_CHUNKS

    def start_gather(c):
        gathers[c] = pltpu.async_copy(
            table_hbm.at[idx_v.at[c]], rows_v.at[c % NBUF], gsem)

    start_gather(0)
    for c in range(N_CHUNKS):
        if c + 1 < N_CHUNKS:
            start_gather(c + 1)
        gathers[c].wait()
        writes[c] = pltpu.async_copy(
            rows_v.at[c % NBUF], out_hbm.at[pl.ds(base + c * CHUNK, CHUNK)],
            wsem)
        if c + NBUF < N_CHUNKS:
            writes[c].wait()
    for c in range(max(0, N_CHUNKS - NBUF), N_CHUNKS):
        writes[c].wait()


def kernel(x_cate, cate_embedding_weight):
    idx = x_cate.astype(jnp.int32).reshape(NW, N_CHUNKS, CHUNK)
    return _gather_kernel(idx, cate_embedding_weight)


# speedup vs baseline: 3.1250x; 3.1250x over previous
import functools

import jax
import jax.numpy as jnp
from jax import lax
from jax.experimental import pallas as pl
from jax.experimental.pallas import tpu as pltpu
from jax.experimental.pallas import tpu_sc as plsc

ROW = 272
BATCH = 16384
NUM_CORES = 2
NUM_SUBCORES = 16
NW = NUM_CORES * NUM_SUBCORES
B_PER_W = BATCH // NW
CHUNK = 32
N_CHUNKS = B_PER_W // CHUNK

_mesh = plsc.VectorSubcoreMesh(core_axis_name="c", subcore_axis_name="s")


@functools.partial(
    pl.kernel,
    mesh=_mesh,
    out_type=jax.ShapeDtypeStruct((BATCH, ROW), jnp.float32),
    scratch_types=[
        pltpu.VMEM((B_PER_W,), jnp.int32),
        pltpu.SemaphoreType.DMA,
    ],
)
def _gather_kernel(idx_hbm, table_hbm, out_hbm, idx_v, gsem):
    wid = lax.axis_index("s") * NUM_CORES + lax.axis_index("c")
    base = wid * B_PER_W
    pltpu.sync_copy(idx_hbm.at[wid], idx_v)

    copies = []
    for c in range(N_CHUNKS):
        for i in range(CHUNK):
            if i % 16 == 0:
                v = idx_v[pl.ds(c * CHUNK + i, 16)]
            r = v[i % 16]
            copies.append(pltpu.async_copy(
                table_hbm.at[pl.ds(r, 1)],
                out_hbm.at[pl.ds(base + c * CHUNK + i, 1)], gsem))
        for cp in copies:
            cp.wait()
        copies = []


def kernel(x_cate, cate_embedding_weight):
    idx = x_cate.astype(jnp.int32).reshape(NW, B_PER_W)
    return _gather_kernel(idx, cate_embedding_weight)


# speedup vs baseline: 5.0097x; 1.6031x over previous
import functools

import jax
import jax.numpy as jnp
from jax import lax
from jax.experimental import pallas as pl
from jax.experimental.pallas import tpu as pltpu
from jax.experimental.pallas import tpu_sc as plsc

ROW = 272
BATCH = 16384
NUM_CORES = 2
NUM_SUBCORES = 16
NW = NUM_CORES * NUM_SUBCORES
B_PER_W = BATCH // NW
SEC = 64
NSEC = B_PER_W // SEC
NBUF = 4

_mesh = plsc.VectorSubcoreMesh(core_axis_name="c", subcore_axis_name="s")


@functools.partial(
    pl.kernel,
    mesh=_mesh,
    out_type=jax.ShapeDtypeStruct((BATCH, ROW), jnp.float32),
    scratch_types=[
        pltpu.VMEM((B_PER_W,), jnp.int32),
        pltpu.VMEM((NBUF, SEC, ROW), jnp.float32),
        pltpu.SemaphoreType.DMA,
        pltpu.SemaphoreType.DMA,
    ],
)
def _gather_kernel(idx_hbm, table_hbm, out_hbm, idx_v, rows_v, gsem, wsem):
    wid = lax.axis_index("s") * NUM_CORES + lax.axis_index("c")
    base = wid * B_PER_W
    pltpu.sync_copy(idx_hbm.at[wid], idx_v)

    writes = [None] * NSEC
    for s in range(NSEC):
        if s >= NBUF:
            writes[s - NBUF].wait()
        b = s % NBUF
        fetches = []
        for i in range(SEC):
            if i % 16 == 0:
                v = idx_v[pl.ds(s * SEC + i, 16)]
            r = v[i % 16]
            fetches.append(pltpu.async_copy(
                table_hbm.at[pl.ds(r, 1)], rows_v.at[b].at[pl.ds(i, 1)],
                gsem))
        for f in fetches:
            f.wait()
        writes[s] = pltpu.async_copy(
            rows_v.at[b], out_hbm.at[pl.ds(base + s * SEC, SEC)], wsem)
    for s in range(NSEC - NBUF, NSEC):
        writes[s].wait()


def kernel(x_cate, cate_embedding_weight):
    idx = x_cate.astype(jnp.int32).reshape(NW, B_PER_W)
    return _gather_kernel(idx, cate_embedding_weight)
